# baseline (speedup 1.0000x reference)
import jax
import jax.numpy as jnp
from jax import lax
from jax.experimental import pallas as pl
from jax.experimental.pallas import tpu as pltpu
from jax.experimental.pallas import tpu_sc as plsc

VOCAB = 1000000
D = 64
B = 16384
C = 50

NC = 2
NS = 16
L = 16
NW = NC * NS
QPW = B // NW
QCHUNK = 16
NCHUNK = QPW // QCHUNK
IDX_MINOR = 100
NSTREAM = (QCHUNK * C) // IDX_MINOR
KD = D // L


def _body(qid_hbm, cid_hbm, table_hbm, out_hbm,
          cidx0, cidx1, qidx0, qidx1, rows0, rows1, qrows0, qrows1,
          scores_v, sem0, sem1):
    wid = lax.axis_index("s") * NC + lax.axis_index("c")
    lanes = lax.iota(jnp.int32, L)
    perms = [lanes ^ sh for sh in (1, 2, 4, 8)]
    cidx = (cidx0, cidx1)
    qidx = (qidx0, qidx1)
    rows = (rows0, rows1)
    qrows = (qrows0, qrows1)
    sems = (sem0, sem1)

    def issue(buf, t):
        qbase = wid * QPW + t * QCHUNK
        cid_off = pl.multiple_of(qbase * C // IDX_MINOR, 8)
        pltpu.sync_copy(cid_hbm.at[pl.ds(cid_off, NSTREAM)], cidx[buf])
        pltpu.sync_copy(qid_hbm.at[pl.ds(pl.multiple_of(qbase, 8), QCHUNK)],
                        qidx[buf])
        for j in range(NSTREAM):
            pltpu.async_copy(
                table_hbm.at[cidx[buf].at[j]],
                rows[buf].at[pl.ds(j * IDX_MINOR, IDX_MINOR)], sems[buf])
        pltpu.async_copy(table_hbm.at[qidx[buf]], qrows[buf], sems[buf])

    def drain(buf):
        for j in range(NSTREAM):
            pltpu.make_async_copy(
                table_hbm.at[cidx[buf].at[j]],
                rows[buf].at[pl.ds(j * IDX_MINOR, IDX_MINOR)],
                sems[buf]).wait()
        pltpu.make_async_copy(table_hbm.at[qidx[buf]], qrows[buf],
                              sems[buf]).wait()

    def compute(buf, t):
        qbase = wid * QPW + t * QCHUNK
        rows_v = rows[buf]
        qrows_v = qrows[buf]

        def q_body(q, carry2):
            qv = [qrows_v[q, pl.ds(k * L, L)] for k in range(KD)]
            for c0 in range(0, C, L):
                n = min(L, C - c0)
                cur = jnp.zeros((L,), jnp.float32)
                for jj in range(n):
                    row = q * C + (c0 + jj)
                    p = qv[0] * rows_v[row, pl.ds(0, L)]
                    for k in range(1, KD):
                        p = p + qv[k] * rows_v[row, pl.ds(k * L, L)]
                    for pm in perms:
                        p = p + jnp.take_along_axis(
                            p, pm, axis=0, mode="promise_in_bounds")
                    cur = jnp.where(lanes == jj, p, cur)
                sig = 1.0 / (1.0 + jnp.exp(-cur))
                scores_v[pl.ds(q * C + c0, L)] = sig
            return carry2

        lax.fori_loop(0, QCHUNK, q_body, 0, unroll=False)
        pltpu.sync_copy(
            scores_v.at[pl.ds(0, QCHUNK * C)],
            out_hbm.at[pl.ds(pl.multiple_of(qbase * C, 8), QCHUNK * C)])

    issue(0, 0)

    def pair_body(tt, carry):
        t0 = 2 * tt
        drain(0)
        issue(1, t0 + 1)
        compute(0, t0)
        drain(1)

        @pl.when(tt + 1 < NCHUNK // 2)
        def _():
            issue(0, t0 + 2)

        compute(1, t0 + 1)
        return carry

    lax.fori_loop(0, NCHUNK // 2, pair_body, 0, unroll=False)


@jax.jit
def _frame(query_id, cand_ids_2d, table):
    kern = pl.kernel(
        _body,
        out_type=jax.ShapeDtypeStruct((B * C,), jnp.float32),
        mesh=plsc.VectorSubcoreMesh(core_axis_name="c", subcore_axis_name="s",
                                    num_cores=NC, num_subcores=NS),
        compiler_params=pltpu.CompilerParams(needs_layout_passes=False,
                                             use_tc_tiling_on_sc=False),
        scratch_types=[
            pltpu.VMEM((NSTREAM, IDX_MINOR), jnp.int32),
            pltpu.VMEM((NSTREAM, IDX_MINOR), jnp.int32),
            pltpu.VMEM((QCHUNK,), jnp.int32),
            pltpu.VMEM((QCHUNK,), jnp.int32),
            pltpu.VMEM((QCHUNK * C, D), jnp.float32),
            pltpu.VMEM((QCHUNK * C, D), jnp.float32),
            pltpu.VMEM((QCHUNK, D), jnp.float32),
            pltpu.VMEM((QCHUNK, D), jnp.float32),
            pltpu.VMEM((QCHUNK * C + L,), jnp.float32),
            pltpu.SemaphoreType.DMA,
            pltpu.SemaphoreType.DMA,
        ],
    )
    return kern(query_id, cand_ids_2d, table)


def kernel(query_id, candidate_hyper_ids, table):
    cand_ids_2d = candidate_hyper_ids.reshape(B * C // IDX_MINOR, IDX_MINOR)
    out = _frame(query_id, cand_ids_2d, table)
    return out.reshape(B, C)


# speedup vs baseline: 1.0269x; 1.0269x over previous
import jax
import jax.numpy as jnp
from jax import lax
from jax.experimental import pallas as pl
from jax.experimental.pallas import tpu as pltpu
from jax.experimental.pallas import tpu_sc as plsc

VOCAB = 1000000
D = 64
B = 16384
C = 50

NC = 2
NS = 16
L = 16
NW = NC * NS
QPW = B // NW
QCHUNK = 16
NCHUNK = QPW // QCHUNK
IDX_MINOR = 100
NSTREAM = (QCHUNK * C) // IDX_MINOR
KD = D // L


def _body(qid_hbm, cid_hbm, table_hbm, out_hbm,
          cidx0, cidx1, qidx0, qidx1, rows0, rows1, qrows0, qrows1,
          scores_v, sem0, sem1):
    wid = lax.axis_index("s") * NC + lax.axis_index("c")
    lanes = lax.iota(jnp.int32, L)
    cidx = (cidx0, cidx1)
    qidx = (qidx0, qidx1)
    rows = (rows0, rows1)
    qrows = (qrows0, qrows1)
    sems = (sem0, sem1)

    def issue(buf, t):
        qbase = wid * QPW + t * QCHUNK
        cid_off = pl.multiple_of(qbase * C // IDX_MINOR, 8)
        pltpu.sync_copy(cid_hbm.at[pl.ds(cid_off, NSTREAM)], cidx[buf])
        pltpu.sync_copy(qid_hbm.at[pl.ds(pl.multiple_of(qbase, 8), QCHUNK)],
                        qidx[buf])
        for j in range(NSTREAM):
            pltpu.async_copy(
                table_hbm.at[cidx[buf].at[j]],
                rows[buf].at[pl.ds(j * IDX_MINOR, IDX_MINOR)], sems[buf])
        pltpu.async_copy(table_hbm.at[qidx[buf]], qrows[buf], sems[buf])

    def drain(buf):
        for j in range(NSTREAM):
            pltpu.make_async_copy(
                table_hbm.at[cidx[buf].at[j]],
                rows[buf].at[pl.ds(j * IDX_MINOR, IDX_MINOR)],
                sems[buf]).wait()
        pltpu.make_async_copy(table_hbm.at[qidx[buf]], qrows[buf],
                              sems[buf]).wait()

    def compute(buf, t):
        qbase = wid * QPW + t * QCHUNK
        rows_v = rows[buf]
        qrows_v = qrows[buf]

        def q_body(q, carry2):
            qv = [qrows_v[q, pl.ds(k * L, L)] for k in range(KD)]
            for c0 in range(0, C, L):
                n = min(L, C - c0)
                cur = jnp.zeros((L,), jnp.float32)
                for jj in range(n):
                    row = q * C + (c0 + jj)
                    p = qv[0] * rows_v[row, pl.ds(0, L)]
                    for k in range(1, KD):
                        p = p + qv[k] * rows_v[row, pl.ds(k * L, L)]
                    s = jnp.sum(p)
                    cur = jnp.where(lanes == jj, s, cur)
                sig = 1.0 / (1.0 + jnp.exp(-cur))
                scores_v[pl.ds(q * C + c0, L)] = sig
            return carry2

        lax.fori_loop(0, QCHUNK, q_body, 0, unroll=False)
        pltpu.sync_copy(
            scores_v.at[pl.ds(0, QCHUNK * C)],
            out_hbm.at[pl.ds(pl.multiple_of(qbase * C, 8), QCHUNK * C)])

    issue(0, 0)

    def pair_body(tt, carry):
        t0 = 2 * tt
        drain(0)
        issue(1, t0 + 1)
        compute(0, t0)
        drain(1)

        @pl.when(tt + 1 < NCHUNK // 2)
        def _():
            issue(0, t0 + 2)

        compute(1, t0 + 1)
        return carry

    lax.fori_loop(0, NCHUNK // 2, pair_body, 0, unroll=False)


@jax.jit
def _frame(query_id, cand_ids_2d, table):
    kern = pl.kernel(
        _body,
        out_type=jax.ShapeDtypeStruct((B * C,), jnp.float32),
        mesh=plsc.VectorSubcoreMesh(core_axis_name="c", subcore_axis_name="s",
                                    num_cores=NC, num_subcores=NS),
        compiler_params=pltpu.CompilerParams(needs_layout_passes=False,
                                             use_tc_tiling_on_sc=False),
        scratch_types=[
            pltpu.VMEM((NSTREAM, IDX_MINOR), jnp.int32),
            pltpu.VMEM((NSTREAM, IDX_MINOR), jnp.int32),
            pltpu.VMEM((QCHUNK,), jnp.int32),
            pltpu.VMEM((QCHUNK,), jnp.int32),
            pltpu.VMEM((QCHUNK * C, D), jnp.float32),
            pltpu.VMEM((QCHUNK * C, D), jnp.float32),
            pltpu.VMEM((QCHUNK, D), jnp.float32),
            pltpu.VMEM((QCHUNK, D), jnp.float32),
            pltpu.VMEM((QCHUNK * C + L,), jnp.float32),
            pltpu.SemaphoreType.DMA,
            pltpu.SemaphoreType.DMA,
        ],
    )
    return kern(query_id, cand_ids_2d, table)


def kernel(query_id, candidate_hyper_ids, table):
    cand_ids_2d = candidate_hyper_ids.reshape(B * C // IDX_MINOR, IDX_MINOR)
    out = _frame(query_id, cand_ids_2d, table)
    return out.reshape(B, C)
